# Initial kernel scaffold; baseline (speedup 1.0000x reference)
import jax
import jax.numpy as jnp
from jax.experimental import pallas as pl


def kernel(queries, keys):
    raise NotImplementedError("write your pallas kernel here")


# speedup vs baseline: 1.8809x; 1.8809x over previous
import jax
import jax.numpy as jnp
from jax.experimental import pallas as pl
from jax.experimental.pallas import tpu as pltpu

Q = 1024
D = 64
K_TOTAL = 100000
KB = 2000
NBLK = K_TOTAL // KB


def _fused_knn_kernel(q_ref, k_ref, bs_ref, bi_ref, best_ref, idx_ref):
    i = pl.program_id(0)

    @pl.when(i == 0)
    def _init():
        best_ref[...] = jnp.full((Q,), -jnp.inf, jnp.float32)
        idx_ref[...] = jnp.zeros((Q,), jnp.int32)

    s = jax.lax.dot_general(
        q_ref[...], k_ref[...], (((1,), (1,)), ((), ())),
        preferred_element_type=jnp.float32)

    bm = jnp.max(s, axis=1)
    col = i * KB + jax.lax.broadcasted_iota(jnp.int32, (1, KB), 1)
    cand = jnp.where(s == bm[:, None], col, jnp.int32(0x7FFFFFFF))
    bi = jnp.min(cand, axis=1)

    prev = best_ref[...]
    upd = bm > prev
    best_ref[...] = jnp.where(upd, bm, prev)
    idx_ref[...] = jnp.where(upd, bi, idx_ref[...])

    @pl.when(i == NBLK - 1)
    def _fin():
        bs_ref[...] = best_ref[...]
        bi_ref[...] = idx_ref[...]


@jax.jit
def kernel(queries, keys):
    qn = queries / jnp.clip(
        jnp.linalg.norm(queries, axis=1, keepdims=True), 1e-12, None)
    kn = keys / jnp.clip(
        jnp.linalg.norm(keys, axis=1, keepdims=True), 1e-12, None)
    qb = qn.astype(jnp.bfloat16)
    kb = kn.astype(jnp.bfloat16)

    best, idx = pl.pallas_call(
        _fused_knn_kernel,
        grid=(NBLK,),
        in_specs=[
            pl.BlockSpec((Q, D), lambda i: (0, 0)),
            pl.BlockSpec((KB, D), lambda i: (i, 0)),
        ],
        out_specs=[
            pl.BlockSpec((Q,), lambda i: (0,)),
            pl.BlockSpec((Q,), lambda i: (0,)),
        ],
        out_shape=[
            jax.ShapeDtypeStruct((Q,), jnp.float32),
            jax.ShapeDtypeStruct((Q,), jnp.int32),
        ],
        scratch_shapes=[
            pltpu.VMEM((Q,), jnp.float32),
            pltpu.VMEM((Q,), jnp.int32),
        ],
        compiler_params=pltpu.CompilerParams(
            dimension_semantics=("arbitrary",),
        ),
    )(qb, kb)
    return best, idx


# speedup vs baseline: 2.4002x; 1.2761x over previous
import jax
import jax.numpy as jnp
from jax.experimental import pallas as pl
from jax.experimental.pallas import tpu as pltpu

Q = 1024
D = 64
K_TOTAL = 100000
KB = 2000
NBLK = K_TOTAL // KB


def _fused_knn_kernel(q_ref, k_ref, bs_ref, bi_ref, best_ref, idx_ref):
    i = pl.program_id(0)

    @pl.when(i == 0)
    def _init():
        best_ref[...] = jnp.full((Q,), -jnp.inf, jnp.float32)
        idx_ref[...] = jnp.zeros((Q,), jnp.float32)

    s = jax.lax.dot_general(
        k_ref[...], q_ref[...], (((1,), (1,)), ((), ())),
        preferred_element_type=jnp.float32)

    bm = jnp.max(s, axis=0)
    row = (jnp.float32(i * KB)
           + jax.lax.broadcasted_iota(jnp.int32, (KB, 1), 0).astype(jnp.float32))
    cand = jnp.where(s == bm[None, :], row, jnp.inf)
    bi = jnp.min(cand, axis=0)

    prev = best_ref[...]
    upd = bm > prev
    best_ref[...] = jnp.where(upd, bm, prev)
    idx_ref[...] = jnp.where(upd, bi, idx_ref[...])

    @pl.when(i == NBLK - 1)
    def _fin():
        bs_ref[...] = best_ref[...]
        bi_ref[...] = idx_ref[...].astype(jnp.int32)


@jax.jit
def kernel(queries, keys):
    qn = queries / jnp.clip(
        jnp.linalg.norm(queries, axis=1, keepdims=True), 1e-12, None)
    kn = keys / jnp.clip(
        jnp.linalg.norm(keys, axis=1, keepdims=True), 1e-12, None)
    qb = qn.astype(jnp.bfloat16)
    kb = kn.astype(jnp.bfloat16)

    best, idx = pl.pallas_call(
        _fused_knn_kernel,
        grid=(NBLK,),
        in_specs=[
            pl.BlockSpec((Q, D), lambda i: (0, 0)),
            pl.BlockSpec((KB, D), lambda i: (i, 0)),
        ],
        out_specs=[
            pl.BlockSpec((Q,), lambda i: (0,)),
            pl.BlockSpec((Q,), lambda i: (0,)),
        ],
        out_shape=[
            jax.ShapeDtypeStruct((Q,), jnp.float32),
            jax.ShapeDtypeStruct((Q,), jnp.int32),
        ],
        scratch_shapes=[
            pltpu.VMEM((Q,), jnp.float32),
            pltpu.VMEM((Q,), jnp.float32),
        ],
        compiler_params=pltpu.CompilerParams(
            dimension_semantics=("arbitrary",),
        ),
    )(qb, kb)
    return best, idx


# speedup vs baseline: 2.4677x; 1.0281x over previous
import jax
import jax.numpy as jnp
from jax.experimental import pallas as pl
from jax.experimental.pallas import tpu as pltpu

Q = 1024
D = 64
K_TOTAL = 100000
KB = 2000
NBLK = K_TOTAL // KB


def _fused_knn_kernel(q_ref, k_ref, n_ref, bs_ref, bi_ref, best_ref, idx_ref):
    i = pl.program_id(0)

    @pl.when(i == 0)
    def _init():
        best_ref[...] = jnp.full((Q,), -jnp.inf, jnp.float32)
        idx_ref[...] = jnp.zeros((Q,), jnp.float32)

    kb = (k_ref[...] / n_ref[...]).astype(jnp.bfloat16)
    s = jax.lax.dot_general(
        kb, q_ref[...], (((1,), (1,)), ((), ())),
        preferred_element_type=jnp.float32)

    bm = jnp.max(s, axis=0)
    row = (jnp.float32(i * KB)
           + jax.lax.broadcasted_iota(jnp.int32, (KB, 1), 0).astype(jnp.float32))
    cand = jnp.where(s == bm[None, :], row, jnp.inf)
    bi = jnp.min(cand, axis=0)

    prev = best_ref[...]
    upd = bm > prev
    best_ref[...] = jnp.where(upd, bm, prev)
    idx_ref[...] = jnp.where(upd, bi, idx_ref[...])

    @pl.when(i == NBLK - 1)
    def _fin():
        bs_ref[...] = best_ref[...]
        bi_ref[...] = idx_ref[...].astype(jnp.int32)


@jax.jit
def kernel(queries, keys):
    qn = queries / jnp.clip(
        jnp.linalg.norm(queries, axis=1, keepdims=True), 1e-12, None)
    qb = qn.astype(jnp.bfloat16)
    knorm = jnp.clip(
        jnp.linalg.norm(keys, axis=1, keepdims=True), 1e-12, None)

    best, idx = pl.pallas_call(
        _fused_knn_kernel,
        grid=(NBLK,),
        in_specs=[
            pl.BlockSpec((Q, D), lambda i: (0, 0)),
            pl.BlockSpec((KB, D), lambda i: (i, 0)),
            pl.BlockSpec((KB, 1), lambda i: (i, 0)),
        ],
        out_specs=[
            pl.BlockSpec((Q,), lambda i: (0,)),
            pl.BlockSpec((Q,), lambda i: (0,)),
        ],
        out_shape=[
            jax.ShapeDtypeStruct((Q,), jnp.float32),
            jax.ShapeDtypeStruct((Q,), jnp.int32),
        ],
        scratch_shapes=[
            pltpu.VMEM((Q,), jnp.float32),
            pltpu.VMEM((Q,), jnp.float32),
        ],
        compiler_params=pltpu.CompilerParams(
            dimension_semantics=("arbitrary",),
        ),
    )(qb, keys, knorm)
    return best, idx


# speedup vs baseline: 2.9890x; 1.2112x over previous
import jax
import jax.numpy as jnp
from jax.experimental import pallas as pl
from jax.experimental.pallas import tpu as pltpu

Q = 1024
D = 64
K_TOTAL = 100000
KB = 2000
NBLK = K_TOTAL // KB


def _fused_knn_kernel(q_ref, k_ref, n_ref, bs_ref, bi_ref, best_ref, idx_ref):
    i = pl.program_id(0)

    @pl.when(i == 0)
    def _init():
        best_ref[...] = jnp.full((Q,), -jnp.inf, jnp.float32)
        idx_ref[...] = jnp.zeros((Q,), jnp.float32)

    kb = (k_ref[...] / n_ref[0].reshape(KB, 1)).astype(jnp.bfloat16)
    s = jax.lax.dot_general(
        kb, q_ref[...], (((1,), (1,)), ((), ())),
        preferred_element_type=jnp.float32)

    bm = jnp.max(s, axis=0)
    row = (jnp.float32(i * KB)
           + jax.lax.broadcasted_iota(jnp.int32, (KB, 1), 0).astype(jnp.float32))
    cand = jnp.where(s == bm[None, :], row, jnp.inf)
    bi = jnp.min(cand, axis=0)

    prev = best_ref[...]
    upd = bm > prev
    best_ref[...] = jnp.where(upd, bm, prev)
    idx_ref[...] = jnp.where(upd, bi, idx_ref[...])

    @pl.when(i == NBLK - 1)
    def _fin():
        bs_ref[...] = best_ref[...]
        bi_ref[...] = idx_ref[...].astype(jnp.int32)


@jax.jit
def kernel(queries, keys):
    qn = queries / jnp.clip(
        jnp.linalg.norm(queries, axis=1, keepdims=True), 1e-12, None)
    qb = qn.astype(jnp.bfloat16)
    knorm = jnp.clip(
        jnp.linalg.norm(keys, axis=1), 1e-12, None).reshape(NBLK, 1, KB)

    best, idx = pl.pallas_call(
        _fused_knn_kernel,
        grid=(NBLK,),
        in_specs=[
            pl.BlockSpec((Q, D), lambda i: (0, 0)),
            pl.BlockSpec((KB, D), lambda i: (i, 0)),
            pl.BlockSpec((1, 1, KB), lambda i: (i, 0, 0)),
        ],
        out_specs=[
            pl.BlockSpec((Q,), lambda i: (0,)),
            pl.BlockSpec((Q,), lambda i: (0,)),
        ],
        out_shape=[
            jax.ShapeDtypeStruct((Q,), jnp.float32),
            jax.ShapeDtypeStruct((Q,), jnp.int32),
        ],
        scratch_shapes=[
            pltpu.VMEM((Q,), jnp.float32),
            pltpu.VMEM((Q,), jnp.float32),
        ],
        compiler_params=pltpu.CompilerParams(
            dimension_semantics=("arbitrary",),
        ),
    )(qb, keys, knorm)
    return best, idx


# speedup vs baseline: 3.4085x; 1.1404x over previous
import jax
import jax.numpy as jnp
from jax.experimental import pallas as pl
from jax.experimental.pallas import tpu as pltpu

Q = 1024
D = 64
K_TOTAL = 100000
KB = 2000
NBLK = K_TOTAL // KB


def _fused_knn_kernel(q_ref, k_ref, n_ref, bs_ref, bi_ref, best_ref, idx_ref,
                      iota_ref):
    i = pl.program_id(0)

    @pl.when(i == 0)
    def _init():
        best_ref[...] = jnp.full((Q,), -jnp.inf, jnp.float32)
        idx_ref[...] = jnp.zeros((Q,), jnp.float32)
        iota_ref[...] = jax.lax.broadcasted_iota(
            jnp.int32, (KB, 1), 0).astype(jnp.float32)

    kb = (k_ref[...] / n_ref[0].reshape(KB, 1)).astype(jnp.bfloat16)
    s = jax.lax.dot_general(
        kb, q_ref[...], (((1,), (1,)), ((), ())),
        preferred_element_type=jnp.float32)

    bm = jnp.max(s, axis=0)
    cand = jnp.where(s == bm[None, :], iota_ref[...], jnp.inf)
    bi = jnp.min(cand, axis=0) + jnp.float32(KB) * i

    prev = best_ref[...]
    upd = bm > prev
    best_ref[...] = jnp.where(upd, bm, prev)
    idx_ref[...] = jnp.where(upd, bi, idx_ref[...])

    @pl.when(i == NBLK - 1)
    def _fin():
        bs_ref[...] = best_ref[...]
        bi_ref[...] = idx_ref[...].astype(jnp.int32)


@jax.jit
def kernel(queries, keys):
    qn = queries / jnp.clip(
        jnp.linalg.norm(queries, axis=1, keepdims=True), 1e-12, None)
    qb = qn.astype(jnp.bfloat16)
    knorm = jnp.clip(
        jnp.linalg.norm(keys, axis=1), 1e-12, None).reshape(NBLK, 1, KB)

    best, idx = pl.pallas_call(
        _fused_knn_kernel,
        grid=(NBLK,),
        in_specs=[
            pl.BlockSpec((Q, D), lambda i: (0, 0)),
            pl.BlockSpec((KB, D), lambda i: (i, 0)),
            pl.BlockSpec((1, 1, KB), lambda i: (i, 0, 0)),
        ],
        out_specs=[
            pl.BlockSpec((Q,), lambda i: (0,)),
            pl.BlockSpec((Q,), lambda i: (0,)),
        ],
        out_shape=[
            jax.ShapeDtypeStruct((Q,), jnp.float32),
            jax.ShapeDtypeStruct((Q,), jnp.int32),
        ],
        scratch_shapes=[
            pltpu.VMEM((Q,), jnp.float32),
            pltpu.VMEM((Q,), jnp.float32),
            pltpu.VMEM((KB, 1), jnp.float32),
        ],
        compiler_params=pltpu.CompilerParams(
            dimension_semantics=("arbitrary",),
        ),
    )(qb, keys, knorm)
    return best, idx


# speedup vs baseline: 3.5721x; 1.0480x over previous
import jax
import jax.numpy as jnp
from jax.experimental import pallas as pl
from jax.experimental.pallas import tpu as pltpu

Q = 1024
D = 64
K_TOTAL = 100000
KB = 4000
NBLK = K_TOTAL // KB


def _fused_knn_kernel(q_ref, k_ref, n_ref, bs_ref, bi_ref, best_ref, idx_ref,
                      iota_ref):
    i = pl.program_id(0)

    @pl.when(i == 0)
    def _init():
        best_ref[...] = jnp.full((Q,), -jnp.inf, jnp.float32)
        idx_ref[...] = jnp.zeros((Q,), jnp.float32)
        iota_ref[...] = jax.lax.broadcasted_iota(
            jnp.int32, (KB, 1), 0).astype(jnp.float32)

    kb = (k_ref[...] / n_ref[0].reshape(KB, 1)).astype(jnp.bfloat16)
    s = jax.lax.dot_general(
        kb, q_ref[...], (((1,), (1,)), ((), ())),
        preferred_element_type=jnp.float32)

    bm = jnp.max(s, axis=0)
    cand = jnp.where(s == bm[None, :], iota_ref[...], jnp.inf)
    bi = jnp.min(cand, axis=0) + jnp.float32(KB) * i

    prev = best_ref[...]
    upd = bm > prev
    best_ref[...] = jnp.where(upd, bm, prev)
    idx_ref[...] = jnp.where(upd, bi, idx_ref[...])

    @pl.when(i == NBLK - 1)
    def _fin():
        bs_ref[...] = best_ref[...]
        bi_ref[...] = idx_ref[...].astype(jnp.int32)


@jax.jit
def kernel(queries, keys):
    qn = queries / jnp.clip(
        jnp.linalg.norm(queries, axis=1, keepdims=True), 1e-12, None)
    qb = qn.astype(jnp.bfloat16)
    knorm = jnp.clip(
        jnp.linalg.norm(keys, axis=1), 1e-12, None).reshape(NBLK, 1, KB)

    best, idx = pl.pallas_call(
        _fused_knn_kernel,
        grid=(NBLK,),
        in_specs=[
            pl.BlockSpec((Q, D), lambda i: (0, 0)),
            pl.BlockSpec((KB, D), lambda i: (i, 0)),
            pl.BlockSpec((1, 1, KB), lambda i: (i, 0, 0)),
        ],
        out_specs=[
            pl.BlockSpec((Q,), lambda i: (0,)),
            pl.BlockSpec((Q,), lambda i: (0,)),
        ],
        out_shape=[
            jax.ShapeDtypeStruct((Q,), jnp.float32),
            jax.ShapeDtypeStruct((Q,), jnp.int32),
        ],
        scratch_shapes=[
            pltpu.VMEM((Q,), jnp.float32),
            pltpu.VMEM((Q,), jnp.float32),
            pltpu.VMEM((KB, 1), jnp.float32),
        ],
        compiler_params=pltpu.CompilerParams(
            dimension_semantics=("arbitrary",),
        ),
    )(qb, keys, knorm)
    return best, idx


# speedup vs baseline: 4.7975x; 1.3431x over previous
import jax
import jax.numpy as jnp
from jax.experimental import pallas as pl
from jax.experimental.pallas import tpu as pltpu

Q = 1024
D = 64
K_TOTAL = 100000
KB = 4000
NBLK = K_TOTAL // KB


def _fused_knn_kernel(q_ref, k_ref, n_ref, bs_ref, bi_ref, best_ref, idx_ref,
                      iota_ref):
    i = pl.program_id(0)

    @pl.when(i == 0)
    def _init():
        best_ref[...] = jnp.full((Q,), -jnp.inf, jnp.float32)
        idx_ref[...] = jnp.zeros((Q,), jnp.float32)
        iota_ref[...] = jax.lax.broadcasted_iota(
            jnp.int32, (KB, 1), 0).astype(jnp.float32)

    kb = (k_ref[...] / n_ref[0].reshape(KB, 1)).astype(jnp.bfloat16)
    s = jax.lax.dot_general(
        kb, q_ref[...], (((1,), (1,)), ((), ())),
        preferred_element_type=jnp.float32)

    bm = jnp.max(s, axis=0)
    bi = (jnp.argmax(s, axis=0).astype(jnp.float32)
          + jnp.float32(KB) * i)

    prev = best_ref[...]
    upd = bm > prev
    best_ref[...] = jnp.where(upd, bm, prev)
    idx_ref[...] = jnp.where(upd, bi, idx_ref[...])

    @pl.when(i == NBLK - 1)
    def _fin():
        bs_ref[...] = best_ref[...]
        bi_ref[...] = idx_ref[...].astype(jnp.int32)


@jax.jit
def kernel(queries, keys):
    qn = queries / jnp.clip(
        jnp.linalg.norm(queries, axis=1, keepdims=True), 1e-12, None)
    qb = qn.astype(jnp.bfloat16)
    knorm = jnp.clip(
        jnp.linalg.norm(keys, axis=1), 1e-12, None).reshape(NBLK, 1, KB)

    best, idx = pl.pallas_call(
        _fused_knn_kernel,
        grid=(NBLK,),
        in_specs=[
            pl.BlockSpec((Q, D), lambda i: (0, 0)),
            pl.BlockSpec((KB, D), lambda i: (i, 0)),
            pl.BlockSpec((1, 1, KB), lambda i: (i, 0, 0)),
        ],
        out_specs=[
            pl.BlockSpec((Q,), lambda i: (0,)),
            pl.BlockSpec((Q,), lambda i: (0,)),
        ],
        out_shape=[
            jax.ShapeDtypeStruct((Q,), jnp.float32),
            jax.ShapeDtypeStruct((Q,), jnp.int32),
        ],
        scratch_shapes=[
            pltpu.VMEM((Q,), jnp.float32),
            pltpu.VMEM((Q,), jnp.float32),
            pltpu.VMEM((KB, 1), jnp.float32),
        ],
        compiler_params=pltpu.CompilerParams(
            dimension_semantics=("arbitrary",),
        ),
    )(qb, keys, knorm)
    return best, idx
